# baseline (speedup 1.0000x reference)
import functools

import jax
import jax.numpy as jnp
from jax import lax
from jax.experimental import pallas as pl
from jax.experimental.pallas import tpu as pltpu
from jax.experimental.pallas import tpu_sc as plsc
from jax.experimental import layout as jlayout

PAD = 4
L = 16
LANES = 128
RING = 2 * PAD + 2


def _make_crop_kernel(B, C, H, W):
    assert B % (2 * LANES) == 0
    n_sub = LANES // L

    n_planes = 2 * C * H

    info = plsc.get_sparse_core_info()
    nw = info.num_cores * info.num_subcores

    mesh = plsc.VectorSubcoreMesh(core_axis_name="c", subcore_axis_name="s")

    @functools.partial(
        pl.kernel,
        mesh=mesh,
        out_type=jax.ShapeDtypeStruct((C, H, W, B), jnp.float32),
        compiler_params=pltpu.CompilerParams(needs_layout_passes=False),
        scratch_types=[
            pltpu.VMEM((B,), jnp.int32),
            pltpu.VMEM((B,), jnp.int32),
            pltpu.VMEM((RING * W, LANES), jnp.float32),
            pltpu.VMEM((W, LANES), jnp.float32),
            pltpu.VMEM((W, LANES), jnp.float32),
            pltpu.SemaphoreType.DMA,
            pltpu.SemaphoreType.DMA,
            pltpu.SemaphoreType.DMA,
        ],
    )
    def crop_kernel(y_hbm, dh_hbm, dw_hbm, out_hbm,
                    dh_v, dw_v, ring_v, ob0, ob1, si, so0, so1):
        wid = lax.axis_index("s") * info.num_cores + lax.axis_index("c")
        q0 = lax.div(wid * n_planes, nw)
        q1 = lax.div((wid + 1) * n_planes, nw)

        pltpu.sync_copy(dh_hbm, dh_v)
        pltpu.sync_copy(dw_hbm, dw_v)

        iota = lax.iota(jnp.int32, L)

        def in_plane_copy(c, hsrc, half, slot):
            return pltpu.make_async_copy(
                y_hbm.at[c, hsrc, :, pl.ds(half * LANES, LANES)],
                ring_v.at[pl.ds(slot * W, W)], si)

        def out_plane_copy(c, h, half, obuf, so):
            return pltpu.make_async_copy(
                obuf,
                out_hbm.at[c, h, :, pl.ds(half * LANES, LANES)], so)

        def step(i, carry):
            pending, since = carry
            q = q0 + i
            half = lax.div(q, C * H)
            r = lax.rem(q, C * H)
            c = lax.div(r, H)
            h = lax.rem(r, H)

            refill = jnp.logical_or(h == 0, i == 0)

            @pl.when(jnp.logical_and(refill, pending >= 1))
            def _():
                in_plane_copy(0, 0, 0, 0).wait()

            @pl.when(jnp.logical_and(refill, pending >= 2))
            def _():
                in_plane_copy(0, 0, 0, 0).wait()

            @pl.when(refill)
            def _():
                for k in range(RING):
                    hp = h - PAD + k
                    slot = lax.rem(hp + RING, RING)
                    hs = jnp.clip(hp, 0, H - 1)
                    in_plane_copy(c, hs, half, slot).start()
                for _k in range(RING):
                    in_plane_copy(0, 0, 0, 0).wait()

            @pl.when(jnp.logical_and(~refill, since >= 2))
            def _():
                in_plane_copy(0, 0, 0, 0).wait()

            dhv, dwv, rbase, rbd, lnv = [], [], [], [], []
            for s in range(n_sub):
                bvec = iota + (half * LANES + s * L)
                dhv.append(plsc.load_gather(dh_v, [bvec]))
                dwv.append(plsc.load_gather(dw_v, [bvec]))
                rbase.append(
                    lax.rem(jnp.clip(dhv[s] + h, 0, H - 1), RING) * W)
                rbd.append(rbase[s] + dwv[s])
                lnv.append(iota + (s * L))

            def do_plane(obuf, so, reuse_cond):
                @pl.when(reuse_cond)
                def _():
                    out_plane_copy(0, 0, 0, obuf, so).wait()

                for wa in range(PAD):
                    for s in range(n_sub):
                        wsv = jnp.maximum(dwv[s] + wa, 0)
                        obuf[wa, pl.ds(s * L, L)] = plsc.load_gather(
                            ring_v, [rbase[s] + wsv, lnv[s]])
                for wa in range(W - PAD, W):
                    for s in range(n_sub):
                        wsv = jnp.minimum(dwv[s] + wa, W - 1)
                        obuf[wa, pl.ds(s * L, L)] = plsc.load_gather(
                            ring_v, [rbase[s] + wsv, lnv[s]])

                @plsc.parallel_loop(PAD, W - PAD, 1, unroll=8)
                def w_body(wa):
                    for s in range(n_sub):
                        obuf[wa, pl.ds(s * L, L)] = plsc.load_gather(
                            ring_v, [rbd[s] + wa, lnv[s]])

                out_plane_copy(c, h, half, obuf, so).start()

            even = lax.rem(i, 2) == 0

            @pl.when(even)
            def _():
                do_plane(ob0, so0, i >= 2)

            @pl.when(~even)
            def _():
                do_plane(ob1, so1, i >= 3)

            in_plane_copy(c, jnp.clip(h + RING - PAD, 0, H - 1), half,
                          lax.rem(h + RING - PAD, RING)).start()

            pending2 = jnp.where(
                refill, jnp.int32(1),
                jnp.where(since >= 2, pending, pending + 1))
            since2 = jnp.where(refill, jnp.int32(1), since + 1)
            return pending2, since2

        pending, _ = lax.fori_loop(
            0, q1 - q0, step, (jnp.int32(0), jnp.int32(0)))

        @pl.when(pending >= 1)
        def _():
            in_plane_copy(0, 0, 0, 0).wait()

        @pl.when(pending >= 2)
        def _():
            in_plane_copy(0, 0, 0, 0).wait()

        out_plane_copy(0, 0, 0, ob0, so0).wait()
        out_plane_copy(0, 0, 0, ob1, so1).wait()

    return crop_kernel


def kernel(x):
    B, C, H, W = x.shape

    k = jax.random.key(42)
    k1, k2 = jax.random.split(k)
    crop_h = jax.random.randint(k1, (B,), 0, 2 * PAD + 1)
    crop_w = jax.random.randint(k2, (B,), 0, 2 * PAD + 1)
    dh = (crop_h - PAD).astype(jnp.int32)
    dw = (crop_w - PAD).astype(jnp.int32)

    y = jnp.transpose(x, (1, 2, 3, 0))
    out_y = _make_crop_kernel(B, C, H, W)(y, dh, dw)
    out = jnp.transpose(out_y, (3, 0, 1, 2))
    return jlayout.with_layout_constraint(
        out, jlayout.Layout(major_to_minor=(1, 2, 3, 0))
    )
